# Initial kernel scaffold; baseline (speedup 1.0000x reference)
import jax
import jax.numpy as jnp
from jax.experimental import pallas as pl


def kernel(beta, particle_id, ec_hit_mask):
    raise NotImplementedError("write your pallas kernel here")


# speedup vs baseline: 3.1213x; 3.1213x over previous
import functools

import jax
import jax.numpy as jnp
from jax import lax
from jax.experimental import pallas as pl
from jax.experimental.pallas import tpu as pltpu
from jax.experimental.pallas import tpu_sc as plsc

_SB = 0.1
_N = 65536
_NBINS = 512
_NC = 2
_NS = 16
_NW = _NC * _NS
_CHUNK = _N // _NW
_VECS = _CHUNK // 16

_mesh = plsc.VectorSubcoreMesh(core_axis_name="c", subcore_axis_name="s")


@functools.partial(
    pl.kernel,
    mesh=_mesh,
    compiler_params=pltpu.CompilerParams(needs_layout_passes=False),
    out_type=(
        jax.ShapeDtypeStruct((_NW, _NBINS), jnp.float32),
        jax.ShapeDtypeStruct((_NW, 32), jnp.float32),
    ),
    scratch_types=[
        pltpu.VMEM((_CHUNK,), jnp.float32),
        pltpu.VMEM((_CHUNK,), jnp.int32),
        pltpu.VMEM((16 * _NBINS,), jnp.float32),
        pltpu.VMEM((_NBINS,), jnp.float32),
        pltpu.VMEM((32,), jnp.float32),
    ],
)
def _sc_segmax(beta_hbm, pid_hbm, mx_out, nz_out, beta_v, pid_v, bins_v, red_v, nz_v):
    wid = lax.axis_index("s") * _NC + lax.axis_index("c")
    base = wid * _CHUNK
    pltpu.sync_copy(beta_hbm.at[pl.ds(base, _CHUNK)], beta_v)
    pltpu.sync_copy(pid_hbm.at[pl.ds(base, _CHUNK)], pid_v)

    lane = lax.broadcasted_iota(jnp.int32, (16,), 0)
    neg = jnp.full((16,), -1.0, jnp.float32)
    zero = jnp.zeros((16,), jnp.float32)

    def init_body(i, carry):
        bins_v[pl.ds(i * 16, 16)] = neg
        return carry

    lax.fori_loop(0, 16 * _NBINS // 16, init_body, 0)

    lane_base = lane * _NBINS

    def body(j, carry):
        nsum, ncnt = carry
        pidv = pid_v[pl.ds(j * 16, 16)]
        betav = beta_v[pl.ds(j * 16, 16)]
        flat = lane_base + pidv
        cur = plsc.load_gather(bins_v, [flat])
        plsc.store_scatter(bins_v, [flat], jnp.maximum(cur, betav))
        isnz = pidv == 0
        nsum = nsum + jnp.where(isnz, betav, 0.0)
        ncnt = ncnt + jnp.where(isnz, 1.0, 0.0)
        return nsum, ncnt

    nsum, ncnt = lax.fori_loop(0, _VECS, body, (zero, zero))

    def red_body(c, carry):
        acc = bins_v[pl.ds(c * 16, 16)]

        def rrow(l, a):
            return jnp.maximum(a, bins_v[pl.ds(l * _NBINS + c * 16, 16)])

        acc = lax.fori_loop(1, 16, rrow, acc)
        red_v[pl.ds(c * 16, 16)] = acc
        return carry

    lax.fori_loop(0, _NBINS // 16, red_body, 0)

    nz_v[pl.ds(0, 16)] = nsum
    nz_v[pl.ds(16, 16)] = ncnt

    pltpu.sync_copy(red_v, mx_out.at[wid])
    pltpu.sync_copy(nz_v, nz_out.at[wid])


def _merge_body(mx_ref, nz_ref, o_ref):
    mx = mx_ref[...]
    nz = nz_ref[...]
    colmax = jnp.max(mx, axis=0, keepdims=True)
    binid = lax.broadcasted_iota(jnp.int32, (1, _NBINS), 1)
    pres = jnp.logical_and(colmax >= 0.0, binid > 0)
    ssum = jnp.sum(jnp.where(pres, 1.0 - colmax, 0.0))
    scnt = jnp.sum(pres.astype(jnp.float32))
    nsum = jnp.sum(nz[:, 0:16])
    ncnt = jnp.sum(nz[:, 16:32])
    loss = ssum / scnt
    noise = jnp.where(ncnt > 0.0, _SB * nsum / jnp.maximum(ncnt, 1.0), 0.0)
    o_ref[...] = jnp.broadcast_to(loss + noise, (1, 1))


_merge = pl.pallas_call(
    _merge_body,
    out_shape=jax.ShapeDtypeStruct((1, 1), jnp.float32),
)


@jax.jit
def kernel(beta, particle_id, ec_hit_mask):
    pid = jnp.where(ec_hit_mask, particle_id, 0).astype(jnp.int32)
    mx, nz = _sc_segmax(beta, pid)
    return _merge(mx, nz)[0, 0]
